# baseline (speedup 1.0000x reference)
import functools

import jax
import jax.numpy as jnp
from jax import lax
from jax.experimental import pallas as pl
from jax.experimental.pallas import tpu as pltpu
from jax.experimental.pallas import tpu_sc as plsc

_D = 128
_B = 16384
_NC = 2
_NS = 16
_NW = _NC * _NS
_BPW = _B // _NW
_NJ = _D // 16
_CCH = 96
_NCCH = 4
_GR = _BPW - _CCH * _NCCH

_mesh = plsc.VectorSubcoreMesh(core_axis_name="c", subcore_axis_name="s")


@functools.partial(
    pl.kernel,
    mesh=_mesh,
    out_type=jax.ShapeDtypeStruct((_B, _D), jnp.float32),
    scratch_types=[
        pltpu.VMEM_SHARED((2, _D), jnp.float32),
        pltpu.VMEM((2 * _D,), jnp.float32),
        pltpu.VMEM((_BPW,), jnp.int32),
        pltpu.VMEM((_BPW, _D), jnp.float32),
        pltpu.SemaphoreType.DMA,
        pltpu.SemaphoreType.DMA,
        pltpu.SemaphoreType.DMA,
        pltpu.SemaphoreType.DMA,
        pltpu.SemaphoreType.DMA,
    ],
)
def _select_kernel(table_hbm, tflat_hbm, idx_hbm, out_hbm,
                   table_s, table_v, flags_v, rows_v,
                   sem_t, sem_v, sem_i, sem_g, sem_o):
    wid = lax.axis_index("s") * _NC + lax.axis_index("c")
    base = wid * _BPW
    cp_t = pltpu.async_copy(table_hbm, table_s, sem_t)
    cp_v = pltpu.async_copy(tflat_hbm, table_v, sem_v)
    cp_i = pltpu.async_copy(idx_hbm.at[pl.ds(base, _BPW)], flags_v, sem_i)
    cp_t.wait()
    cp_i.wait()
    gbase = _CCH * _NCCH
    gath = pltpu.async_copy(
        table_s.at[flags_v.at[pl.ds(gbase, _GR)]],
        rows_v.at[pl.ds(gbase, _GR)], sem_g)
    cp_v.wait()
    pres = [table_v[pl.ds(16 * j, 16)] for j in range(_NJ)]
    diff = [table_v[pl.ds(_D + 16 * j, 16)] - pres[j] for j in range(_NJ)]
    lane = [jnp.full((16, 1), l, jnp.int32) for l in range(16)]
    dnums = lax.GatherDimensionNumbers(
        offset_dims=(), collapsed_slice_dims=(0,), start_index_map=(0,))

    outs = []
    for c in range(_NCCH):
        coff = c * _CCH

        def body(g, carry, coff=coff):
            rbase = coff + g * 16
            fv = flags_v[pl.ds(rbase, 16)]
            for l in range(16):
                bl = lax.gather(fv, lane[l], dnums, (1,),
                                mode=lax.GatherScatterMode.PROMISE_IN_BOUNDS)
                f = bl.astype(jnp.float32)
                for j in range(_NJ):
                    rows_v[rbase + l, pl.ds(16 * j, 16)] = pres[j] + f * diff[j]
            return carry

        lax.fori_loop(0, _CCH // 16, body, 0)
        outs.append(pltpu.async_copy(
            rows_v.at[pl.ds(coff, _CCH)],
            out_hbm.at[pl.ds(base + coff, _CCH)], sem_o))
    gath.wait()
    outs.append(pltpu.async_copy(
        rows_v.at[pl.ds(gbase, _GR)],
        out_hbm.at[pl.ds(base + gbase, _GR)], sem_o))
    for o in outs:
        o.wait()


def kernel(is_absent, present_goal_vector, absent_goal_vector):
    table = jnp.stack([present_goal_vector, absent_goal_vector])
    idx = is_absent.astype(jnp.int32)
    out = _select_kernel(table, table.reshape(-1), idx)
    return out[None]


# speedup vs baseline: 1.0702x; 1.0702x over previous
import functools

import jax
import jax.numpy as jnp
from jax import lax
from jax.experimental import pallas as pl
from jax.experimental.pallas import tpu as pltpu
from jax.experimental.pallas import tpu_sc as plsc

_D = 128
_B = 16384
_NC = 2
_NS = 16
_NW = _NC * _NS
_BPW = _B // _NW
_NCH = 4
_CH = _BPW // _NCH

_mesh = plsc.VectorSubcoreMesh(core_axis_name="c", subcore_axis_name="s")


@functools.partial(
    pl.kernel,
    mesh=_mesh,
    out_type=jax.ShapeDtypeStruct((_B, _D), jnp.float32),
    scratch_types=[
        pltpu.VMEM_SHARED((_NS, 2, _D), jnp.float32),
        pltpu.VMEM((_BPW,), jnp.int32),
        pltpu.VMEM((_BPW, _D), jnp.float32),
        pltpu.SemaphoreType.DMA,
        pltpu.SemaphoreType.DMA,
        pltpu.SemaphoreType.DMA,
        pltpu.SemaphoreType.DMA,
        pltpu.SemaphoreType.DMA,
        pltpu.SemaphoreType.DMA,
        pltpu.SemaphoreType.DMA,
        pltpu.SemaphoreType.DMA,
        pltpu.SemaphoreType.DMA,
        pltpu.SemaphoreType.DMA,
    ],
)
def _gather_kernel(table_hbm, idx_hbm, out_hbm, table_s, flags_v, rows_v,
                   sem_t, sem_o, i0, i1, i2, i3, g0, g1, g2, g3):
    cid = lax.axis_index("c")
    sid = lax.axis_index("s")
    wid = sid * _NC + cid
    base = wid * _BPW
    isem = [i0, i1, i2, i3]
    gsem = [g0, g1, g2, g3]
    cp_t = pltpu.async_copy(table_hbm, table_s.at[sid], sem_t)
    icps = [pltpu.async_copy(idx_hbm.at[pl.ds(base + k * _CH, _CH)],
                             flags_v.at[pl.ds(k * _CH, _CH)], isem[k])
            for k in range(_NCH)]
    cp_t.wait()
    gaths = []
    for k in range(_NCH):
        icps[k].wait()
        gaths.append(pltpu.async_copy(
            table_s.at[sid].at[flags_v.at[pl.ds(k * _CH, _CH)]],
            rows_v.at[pl.ds(k * _CH, _CH)], gsem[k]))
    outs = []
    for k in range(_NCH):
        gaths[k].wait()
        outs.append(pltpu.async_copy(
            rows_v.at[pl.ds(k * _CH, _CH)],
            out_hbm.at[pl.ds(base + k * _CH, _CH)], sem_o))
    for o in outs:
        o.wait()


def kernel(is_absent, present_goal_vector, absent_goal_vector):
    table = jnp.stack([present_goal_vector, absent_goal_vector])
    idx = is_absent.astype(jnp.int32)
    out = _gather_kernel(table, idx)
    return out[None]


# speedup vs baseline: 1.0719x; 1.0016x over previous
import functools

import jax
import jax.numpy as jnp
from jax import lax
from jax.experimental import pallas as pl
from jax.experimental.pallas import tpu as pltpu
from jax.experimental.pallas import tpu_sc as plsc

_D = 128
_B = 16384
_NC = 2
_NS = 16
_NW = _NC * _NS
_BPW = _B // _NW
_NCH = 8
_CH = _BPW // _NCH

_mesh = plsc.VectorSubcoreMesh(core_axis_name="c", subcore_axis_name="s")


@functools.partial(
    pl.kernel,
    mesh=_mesh,
    out_type=jax.ShapeDtypeStruct((_B, _D), jnp.float32),
    scratch_types=[
        pltpu.VMEM_SHARED((_NS, 2, _D), jnp.float32),
        pltpu.VMEM((_BPW,), jnp.int32),
        pltpu.VMEM((_BPW, _D), jnp.float32),
    ] + [pltpu.SemaphoreType.DMA] * 18,
)
def _gather_kernel(table_hbm, idx_hbm, out_hbm, table_s, flags_v, rows_v,
                   sem_t, sem_o, *ksem):
    cid = lax.axis_index("c")
    sid = lax.axis_index("s")
    wid = sid * _NC + cid
    base = wid * _BPW
    isem = list(ksem[:_NCH])
    gsem = list(ksem[_NCH:])
    cp_t = pltpu.async_copy(table_hbm, table_s.at[sid], sem_t)
    icps = [pltpu.async_copy(idx_hbm.at[pl.ds(base + k * _CH, _CH)],
                             flags_v.at[pl.ds(k * _CH, _CH)], isem[k])
            for k in range(_NCH)]
    cp_t.wait()
    gaths = []
    for k in range(_NCH):
        icps[k].wait()
        gaths.append(pltpu.async_copy(
            table_s.at[sid].at[flags_v.at[pl.ds(k * _CH, _CH)]],
            rows_v.at[pl.ds(k * _CH, _CH)], gsem[k]))
    outs = []
    for k in range(_NCH):
        gaths[k].wait()
        outs.append(pltpu.async_copy(
            rows_v.at[pl.ds(k * _CH, _CH)],
            out_hbm.at[pl.ds(base + k * _CH, _CH)], sem_o))
    for o in outs:
        o.wait()


def kernel(is_absent, present_goal_vector, absent_goal_vector):
    table = jnp.stack([present_goal_vector, absent_goal_vector])
    idx = is_absent.astype(jnp.int32)
    out = _gather_kernel(table, idx)
    return out[None]
